# baseline (speedup 1.0000x reference)
import jax
import jax.numpy as jnp
from jax.experimental import pallas as pl
from jax.experimental.pallas import tpu as pltpu

_K = 16
_D = 1024
_d = 64
_BT = 1024

_PREC = jax.lax.Precision.DEFAULT
_PREC_APPLY = jax.lax.Precision.DEFAULT


def _prep_kernel(us_ref, v_ref, ucat_ref, aug_ref):
    a_list = []
    for k in range(_K):
        u = us_ref[k]
        a_list.append(
            jax.lax.dot_general(u, u, (((0,), (0,)), ((), ())),
                                preferred_element_type=jnp.float32,
                                precision=_PREC))
    a = jnp.stack(a_list, axis=0)

    aug_ref[...] = a
    cols_row = jax.lax.broadcasted_iota(jnp.int32, (_K, 1, _d), 2)
    eye = (jax.lax.broadcasted_iota(jnp.int32, (_K, _d, _d), 1)
           == jax.lax.broadcasted_iota(jnp.int32, (_K, _d, _d), 2)
           ).astype(jnp.float32)

    def sweep_step(j, _):
        cur = aug_ref[...]
        row = aug_ref[:, pl.ds(j, 1), :]
        ej_row = (cols_row == j).astype(jnp.float32)
        piv = jnp.sum(row * ej_row, axis=2, keepdims=True)
        row_adj = row - ej_row
        col_adj = jnp.transpose(row_adj, (0, 2, 1))
        aug_ref[...] = cur - col_adj * (row_adj * (1.0 / piv))
        return 0

    jax.lax.fori_loop(0, _d, sweep_step, 0)
    a_inv = 2.0 * eye - aug_ref[...]

    for k in range(_K):
        u = us_ref[k]
        v_ref[k] = jax.lax.dot_general(u, a_inv[k], (((1,), (0,)), ((), ())),
                                       preferred_element_type=jnp.float32,
                                       precision=_PREC)
        ucat_ref[:, k * _d:(k + 1) * _d] = u


def _apply_kernel(x_ref, ucat_ref, v_ref, o_ref, t_ref):
    j = pl.program_id(1)

    @pl.when(j == 0)
    def _encode():
        t_ref[...] = jax.lax.dot_general(
            x_ref[...], ucat_ref[...], (((1,), (0,)), ((), ())),
            preferred_element_type=jnp.float32, precision=_PREC_APPLY)

    t2 = t_ref[:, pl.ds(j * 2 * _d, 2 * _d)]
    for h in range(2):
        tk = jax.lax.slice(t2, (0, h * _d), (t2.shape[0], (h + 1) * _d))
        o_ref[h] = jax.lax.dot_general(tk, v_ref[h], (((1,), (1,)), ((), ())),
                                       preferred_element_type=jnp.float32,
                                       precision=_PREC_APPLY)


def kernel(x, Us):
    batch = x.shape[0]
    n_bt = batch // _BT

    v, u_cat = pl.pallas_call(
        _prep_kernel,
        out_shape=(jax.ShapeDtypeStruct((_K, _D, _d), jnp.float32),
                   jax.ShapeDtypeStruct((_D, _K * _d), jnp.float32)),
        scratch_shapes=[pltpu.VMEM((_K, _d, _d), jnp.float32)],
    )(Us)

    out = pl.pallas_call(
        _apply_kernel,
        grid=(n_bt, _K // 2),
        in_specs=[
            pl.BlockSpec((_BT, _D), lambda i, j: (i, 0)),
            pl.BlockSpec((_D, _K * _d), lambda i, j: (0, 0)),
            pl.BlockSpec((2, _D, _d), lambda i, j: (j, 0, 0)),
        ],
        out_specs=pl.BlockSpec((2, _BT, _D), lambda i, j: (j, i, 0)),
        out_shape=jax.ShapeDtypeStruct((_K, batch, _D), jnp.float32),
        scratch_shapes=[pltpu.VMEM((_BT, _K * _d), jnp.float32)],
    )(x, u_cat, v)
    return out


# speedup vs baseline: 1.1296x; 1.1296x over previous
import jax
import jax.numpy as jnp
from jax.experimental import pallas as pl
from jax.experimental.pallas import tpu as pltpu

_K = 16
_D = 1024
_d = 64
_BT = 256

_PREC = jax.lax.Precision.DEFAULT
_PREC_APPLY = jax.lax.Precision.DEFAULT


def _prep_kernel(us_ref, v_ref, ucat_ref, aug_ref):
    a_list = []
    for k in range(_K):
        u = us_ref[k]
        a_list.append(
            jax.lax.dot_general(u, u, (((0,), (0,)), ((), ())),
                                preferred_element_type=jnp.float32,
                                precision=_PREC))
    a = jnp.stack(a_list, axis=0)

    aug_ref[...] = a
    cols_row = jax.lax.broadcasted_iota(jnp.int32, (_K, 1, _d), 2)
    eye = (jax.lax.broadcasted_iota(jnp.int32, (_K, _d, _d), 1)
           == jax.lax.broadcasted_iota(jnp.int32, (_K, _d, _d), 2)
           ).astype(jnp.float32)

    def sweep_step(j, cur):
        row = aug_ref[:, pl.ds(j, 1), :]
        ej_row = (cols_row == j).astype(jnp.float32)
        piv = jnp.sum(row * ej_row, axis=2, keepdims=True)
        row_adj = row - ej_row
        col_adj = jnp.transpose(row_adj, (0, 2, 1))
        new = cur - col_adj * (row_adj * (1.0 / piv))
        aug_ref[...] = new
        return new

    jax.lax.fori_loop(0, _d, sweep_step, a, unroll=2)
    a_inv = 2.0 * eye - aug_ref[...]

    for k in range(_K):
        u = us_ref[k]
        v_ref[k] = jax.lax.dot_general(u, a_inv[k], (((1,), (0,)), ((), ())),
                                       preferred_element_type=jnp.float32,
                                       precision=_PREC)
        ucat_ref[:, k * _d:(k + 1) * _d] = u


def _apply_kernel(x_ref, ucat_ref, v_ref, o_ref):
    t = jax.lax.dot_general(x_ref[...], ucat_ref[...], (((1,), (0,)), ((), ())),
                            preferred_element_type=jnp.float32,
                            precision=_PREC_APPLY)
    for k in range(_K):
        tk = jax.lax.slice(t, (0, k * _d), (t.shape[0], (k + 1) * _d))
        o_ref[k] = jax.lax.dot_general(tk, v_ref[k], (((1,), (1,)), ((), ())),
                                       preferred_element_type=jnp.float32,
                                       precision=_PREC_APPLY)


def kernel(x, Us):
    batch = x.shape[0]
    n_bt = batch // _BT

    v, u_cat = pl.pallas_call(
        _prep_kernel,
        out_shape=(jax.ShapeDtypeStruct((_K, _D, _d), jnp.float32),
                   jax.ShapeDtypeStruct((_D, _K * _d), jnp.float32)),
        scratch_shapes=[pltpu.VMEM((_K, _d, _d), jnp.float32)],
    )(Us)

    out = pl.pallas_call(
        _apply_kernel,
        grid=(n_bt,),
        in_specs=[
            pl.BlockSpec((_BT, _D), lambda i: (i, 0)),
            pl.BlockSpec((_D, _K * _d), lambda i: (0, 0)),
            pl.BlockSpec((_K, _D, _d), lambda i: (0, 0, 0)),
        ],
        out_specs=pl.BlockSpec((_K, _BT, _D), lambda i: (0, i, 0)),
        out_shape=jax.ShapeDtypeStruct((_K, batch, _D), jnp.float32),
    )(x, u_cat, v)
    return out


# speedup vs baseline: 1.1385x; 1.0079x over previous
import jax
import jax.numpy as jnp
from jax.experimental import pallas as pl
from jax.experimental.pallas import tpu as pltpu

_K = 16
_D = 1024
_d = 64
_BT = 256

_PREC = jax.lax.Precision.DEFAULT
_PREC_APPLY = jax.lax.Precision.DEFAULT


def _prep_kernel(us_ref, v_ref, ucat_ref, aug_ref):
    a_list = []
    for k in range(_K):
        u = us_ref[k]
        a_list.append(
            jax.lax.dot_general(u, u, (((0,), (0,)), ((), ())),
                                preferred_element_type=jnp.float32,
                                precision=_PREC))
    a = jnp.stack(a_list, axis=0)

    aug_ref[...] = a
    cols_row = jax.lax.broadcasted_iota(jnp.int32, (_K, 1, _d), 2)
    eye = (jax.lax.broadcasted_iota(jnp.int32, (_K, _d, _d), 1)
           == jax.lax.broadcasted_iota(jnp.int32, (_K, _d, _d), 2)
           ).astype(jnp.float32)

    def sweep_step(j, _):
        cur = aug_ref[...]
        row = aug_ref[:, pl.ds(j, 1), :]
        ej_row = (cols_row == j).astype(jnp.float32)
        piv = jnp.sum(row * ej_row, axis=2, keepdims=True)
        row_adj = row - ej_row
        col_adj = jnp.transpose(row_adj, (0, 2, 1))
        aug_ref[...] = cur - col_adj * (row_adj * (1.0 / piv))
        return 0

    jax.lax.fori_loop(0, _d, sweep_step, 0)
    a_inv = 2.0 * eye - aug_ref[...]

    for k in range(_K):
        u = us_ref[k]
        v_ref[k] = jax.lax.dot_general(u, a_inv[k], (((1,), (0,)), ((), ())),
                                       preferred_element_type=jnp.float32,
                                       precision=_PREC)
        ucat_ref[:, k * _d:(k + 1) * _d] = u


def _apply_kernel(x_ref, ucat_ref, v_ref, o_ref):
    t = jax.lax.dot_general(x_ref[...], ucat_ref[...], (((1,), (0,)), ((), ())),
                            preferred_element_type=jnp.float32,
                            precision=_PREC_APPLY)
    for k in range(_K):
        tk = jax.lax.slice(t, (0, k * _d), (t.shape[0], (k + 1) * _d))
        o_ref[k] = jax.lax.dot_general(tk, v_ref[k], (((1,), (1,)), ((), ())),
                                       preferred_element_type=jnp.float32,
                                       precision=_PREC_APPLY)


def kernel(x, Us):
    batch = x.shape[0]
    n_bt = batch // _BT

    v, u_cat = pl.pallas_call(
        _prep_kernel,
        out_shape=(jax.ShapeDtypeStruct((_K, _D, _d), jnp.float32),
                   jax.ShapeDtypeStruct((_D, _K * _d), jnp.float32)),
        scratch_shapes=[pltpu.VMEM((_K, _d, _d), jnp.float32)],
    )(Us)

    out = pl.pallas_call(
        _apply_kernel,
        grid=(n_bt,),
        in_specs=[
            pl.BlockSpec((_BT, _D), lambda i: (i, 0)),
            pl.BlockSpec((_D, _K * _d), lambda i: (0, 0)),
            pl.BlockSpec((_K, _D, _d), lambda i: (0, 0, 0)),
        ],
        out_specs=pl.BlockSpec((_K, _BT, _D), lambda i: (0, i, 0)),
        out_shape=jax.ShapeDtypeStruct((_K, batch, _D), jnp.float32),
    )(x, u_cat, v)
    return out
